# baseline (speedup 1.0000x reference)
import functools

import jax
import jax.numpy as jnp
from jax import lax
from jax.experimental import pallas as pl
from jax.experimental.pallas import tpu as pltpu
from jax.experimental.pallas import tpu_sc as plsc

NUM_CORES = 2
NUM_SUBCORES = 16
NUM_WORKERS = NUM_CORES * NUM_SUBCORES

CH = 8
NBUF = 3


@functools.lru_cache(maxsize=None)
def _make_gather(B, D):
    b_per_w = B // NUM_WORKERS
    n_ch = b_per_w // CH
    assert b_per_w * NUM_WORKERS == B and b_per_w % 8 == 0
    assert n_ch * CH == b_per_w and (n_ch - 5) % NBUF == 0 and n_ch >= 8

    mesh = plsc.VectorSubcoreMesh(core_axis_name="c", subcore_axis_name="s")

    @functools.partial(
        pl.kernel,
        mesh=mesh,
        out_type=jax.ShapeDtypeStruct((B, D), jnp.float32),
        scratch_types=[
            pltpu.VMEM((b_per_w,), jnp.int32),
            *[pltpu.VMEM((CH, D), jnp.float32) for _ in range(NBUF)],
            *[pltpu.SemaphoreType.DMA for _ in range(2 * NBUF)],
        ],
    )
    def gather_kernel(table_hbm, idx_hbm, out_hbm, idx_v, *bufs_sems):
        slots = bufs_sems[:NBUF]
        gsem = bufs_sems[NBUF:2 * NBUF]
        osem = bufs_sems[2 * NBUF:3 * NBUF]

        wid = lax.axis_index("s") * NUM_CORES + lax.axis_index("c")
        base = wid * b_per_w
        pltpu.sync_copy(idx_hbm.at[pl.ds(base, b_per_w)], idx_v)

        def g_start(c, s):
            pltpu.async_copy(table_hbm.at[idx_v.at[pl.ds(c * CH, CH)]],
                             slots[s], gsem[s])

        def g_wait(c, s):
            pltpu.make_async_copy(table_hbm.at[idx_v.at[pl.ds(c * CH, CH)]],
                                  slots[s], gsem[s]).wait()

        def s_start(c, s):
            pltpu.async_copy(slots[s], out_hbm.at[pl.ds(base + c * CH, CH)],
                             osem[s])

        def s_wait(c, s):
            pltpu.make_async_copy(slots[s],
                                  out_hbm.at[pl.ds(base + c * CH, CH)],
                                  osem[s]).wait()

        g_start(0, 0)
        g_start(1, 1)
        g_start(2, 2)
        g_wait(0, 0)
        s_start(0, 0)
        g_wait(1, 1)
        s_wait(0, 0)
        g_start(3, 0)
        s_start(1, 1)
        g_wait(2, 2)
        s_wait(1, 1)
        g_start(4, 1)
        s_start(2, 2)

        @pl.loop(3, n_ch - 2, step=NBUF)
        def _(k):
            for b in range(NBUF):
                c = k + b
                s = b
                g_wait(c, s)
                s_wait(c - 1, (s + 2) % NBUF)
                g_start(c + 2, (s + 2) % NBUF)
                s_start(c, s)

        for c in (n_ch - 2, n_ch - 1):
            g_wait(c, c % NBUF)
            s_start(c, c % NBUF)
        for c in (n_ch - 3, n_ch - 2, n_ch - 1):
            s_wait(c, c % NBUF)

    return gather_kernel


def kernel(input_ids, embed_table):
    batch, seq = input_ids.shape
    vocab, d = embed_table.shape
    idx = input_ids.reshape(-1).astype(jnp.int32)
    out = _make_gather(batch * seq, d)(embed_table, idx)
    return out.reshape(batch, seq, d)


# speedup vs baseline: 36.1448x; 36.1448x over previous
import functools

import jax
import jax.numpy as jnp
from jax import lax
from jax.experimental import pallas as pl
from jax.experimental.pallas import tpu as pltpu
from jax.experimental.pallas import tpu_sc as plsc

NUM_CORES = 2
NUM_SUBCORES = 16
NUM_WORKERS = NUM_CORES * NUM_SUBCORES

CH = 8
NBUF = 3


@functools.lru_cache(maxsize=None)
def _make_gather(BATCH, SEQ, D):
    B = BATCH * SEQ
    b_per_w = B // NUM_WORKERS
    n_ch = b_per_w // CH
    w_per_row = SEQ // b_per_w
    assert b_per_w * NUM_WORKERS == B and b_per_w % 8 == 0
    assert w_per_row * b_per_w == SEQ
    assert n_ch * CH == b_per_w and (n_ch - 5) % NBUF == 0 and n_ch >= 8

    mesh = plsc.VectorSubcoreMesh(core_axis_name="c", subcore_axis_name="s")

    @functools.partial(
        pl.kernel,
        mesh=mesh,
        out_type=jax.ShapeDtypeStruct((B, D), jnp.float32),
        scratch_types=[
            pltpu.VMEM((b_per_w,), jnp.int32),
            *[pltpu.VMEM((CH, D), jnp.float32) for _ in range(NBUF)],
            *[pltpu.SemaphoreType.DMA for _ in range(2 * NBUF)],
        ],
    )
    def gather_kernel(table_hbm, idx_hbm, out_hbm, idx_v, *bufs_sems):
        slots = bufs_sems[:NBUF]
        gsem = bufs_sems[NBUF:2 * NBUF]
        osem = bufs_sems[2 * NBUF:3 * NBUF]

        wid = lax.axis_index("s") * NUM_CORES + lax.axis_index("c")
        base = wid * b_per_w
        row = wid // w_per_row
        col = (wid % w_per_row) * b_per_w
        pltpu.sync_copy(idx_hbm.at[row, pl.ds(col, b_per_w)], idx_v)

        def g_start(c, s):
            pltpu.async_copy(table_hbm.at[idx_v.at[pl.ds(c * CH, CH)]],
                             slots[s], gsem[s])

        def g_wait(c, s):
            pltpu.make_async_copy(table_hbm.at[idx_v.at[pl.ds(c * CH, CH)]],
                                  slots[s], gsem[s]).wait()

        def s_start(c, s):
            pltpu.async_copy(slots[s], out_hbm.at[pl.ds(base + c * CH, CH)],
                             osem[s])

        def s_wait(c, s):
            pltpu.make_async_copy(slots[s],
                                  out_hbm.at[pl.ds(base + c * CH, CH)],
                                  osem[s]).wait()

        g_start(0, 0)
        g_start(1, 1)
        g_start(2, 2)
        g_wait(0, 0)
        s_start(0, 0)
        g_wait(1, 1)
        s_wait(0, 0)
        g_start(3, 0)
        s_start(1, 1)
        g_wait(2, 2)
        s_wait(1, 1)
        g_start(4, 1)
        s_start(2, 2)

        @pl.loop(3, n_ch - 2, step=NBUF)
        def _(k):
            for b in range(NBUF):
                c = k + b
                s = b
                g_wait(c, s)
                s_wait(c - 1, (s + 2) % NBUF)
                g_start(c + 2, (s + 2) % NBUF)
                s_start(c, s)

        for c in (n_ch - 2, n_ch - 1):
            g_wait(c, c % NBUF)
            s_start(c, c % NBUF)
        for c in (n_ch - 3, n_ch - 2, n_ch - 1):
            s_wait(c, c % NBUF)

    return gather_kernel


def kernel(input_ids, embed_table):
    batch, seq = input_ids.shape
    vocab, d = embed_table.shape
    out = _make_gather(batch, seq, d)(embed_table,
                                      input_ids.astype(jnp.int32))
    return out.reshape(batch, seq, d)
